# baseline (speedup 1.0000x reference)
import functools

import jax
import jax.numpy as jnp
from jax import lax
from jax.experimental import pallas as pl
from jax.experimental.pallas import tpu as pltpu
from jax.experimental.pallas import tpu_sc as plsc

_GROUPS = 10
_B = 16384
_D = 256
_K = 1024
_CC = (0.5, 0.5, 0.4, 0.4, 0.4, 0.4, 0.8, 0.8, 0.8, 0.8)


_BB = 512
_NB = _B // _BB


def _vq_tc_body(x_ref, w_ref, idx_ref, loss_ref, w2_ref, wsq_ref,
                xw_ref, xsq_ref):
    i = pl.program_id(0)

    @pl.when(i == 0)
    def _():
        w = w_ref[...]
        w2_ref[...] = -2.0 * w
        wsq_ref[...] = jnp.sum(w * w, axis=1)[None, :]
        loss_ref[0, 0] = 0.0

    x = x_ref[...]
    xw_ref[i % 2] = lax.dot_general(
        x, w2_ref[...], (((1,), (1,)), ((), ())),
        preferred_element_type=jnp.float32)
    xsq_ref[i % 2] = jnp.sum(x * x, axis=1, keepdims=True)

    p = (i + 1) % 2
    m2xw = xw_ref[p]
    dist = (xsq_ref[p] + wsq_ref[...]) + m2xw
    dmin = jnp.min(dist, axis=1, keepdims=True)
    iota = lax.broadcasted_iota(jnp.int32, dist.shape, 1).astype(jnp.float32)
    idxf = jnp.min(jnp.where(dist == dmin, iota, float(_K)), axis=1)
    idx_ref[...] = idxf.astype(jnp.int32)
    loss_ref[0, 0] += jnp.where(i > 0, jnp.sum(dmin), 0.0)


def _vq_tc(x, w):
    return pl.pallas_call(
        _vq_tc_body,
        grid=(_NB + 1,),
        in_specs=[
            pl.BlockSpec((_BB, _D), lambda i: (jnp.minimum(i, _NB - 1), 0)),
            pl.BlockSpec((_K, _D), lambda i: (0, 0)),
        ],
        out_specs=[
            pl.BlockSpec((_BB,), lambda i: (jnp.maximum(i - 1, 0),)),
            pl.BlockSpec((1, 1), lambda i: (0, 0), memory_space=pltpu.SMEM),
        ],
        out_shape=[
            jax.ShapeDtypeStruct((_B,), jnp.int32),
            jax.ShapeDtypeStruct((1, 1), jnp.float32),
        ],
        scratch_shapes=[
            pltpu.VMEM((_K, _D), jnp.float32),
            pltpu.VMEM((1, _K), jnp.float32),
            pltpu.VMEM((2, _BB, _K), jnp.float32),
            pltpu.VMEM((2, _BB, 1), jnp.float32),
        ],
    )(x, w)


_NC = 2
_NS = 16
_NW = _NC * _NS
_ROWS = _GROUPS * _B
_RPW = _ROWS // _NW
_CH = 128
_NCH = _RPW // _CH


def _sc_gather_body(idx_hbm, table_hbm, out_hbm,
                    idx0, idx1, rows0, rows1, gsem, wsem0, wsem1):
    wid = lax.axis_index("s") * _NC + lax.axis_index("c")
    base = wid * _RPW
    idx_bufs = (idx0, idx1)
    row_bufs = (rows0, rows1)
    wsems = (wsem0, wsem1)

    def outer(o, _):
        for b in range(2):
            j = o * 2 + b
            start = base + j * _CH
            pltpu.sync_copy(idx_hbm.at[pl.ds(start, _CH)], idx_bufs[b])
            off = (start // _B) * _K
            for s in range(_CH // 16):
                sl = pl.ds(s * 16, 16)
                idx_bufs[b][sl] = idx_bufs[b][sl] + off

            @pl.when(j >= 2)
            def _():
                pltpu.make_async_copy(
                    row_bufs[b], out_hbm.at[pl.ds(start - 2 * _CH, _CH)],
                    wsems[b]).wait()

            pltpu.async_copy(table_hbm.at[idx_bufs[b]], row_bufs[b],
                             gsem).wait()
            pltpu.async_copy(row_bufs[b], out_hbm.at[pl.ds(start, _CH)],
                             wsems[b])
        return _

    lax.fori_loop(0, _NCH // 2, outer, None)
    for b in range(2):
        j = _NCH - 2 + b
        pltpu.make_async_copy(
            row_bufs[b], out_hbm.at[pl.ds(base + j * _CH, _CH)],
            wsems[b]).wait()


def _sc_gather():
    return pl.kernel(
        _sc_gather_body,
        mesh=plsc.VectorSubcoreMesh(core_axis_name="c", subcore_axis_name="s"),
        out_type=jax.ShapeDtypeStruct((_ROWS, _D), jnp.float32),
        scratch_types=[
            pltpu.VMEM((_CH,), jnp.int32),
            pltpu.VMEM((_CH,), jnp.int32),
            pltpu.VMEM((_CH, _D), jnp.float32),
            pltpu.VMEM((_CH, _D), jnp.float32),
            pltpu.SemaphoreType.DMA,
            pltpu.SemaphoreType.DMA,
            pltpu.SemaphoreType.DMA,
        ],
    )


def kernel(head_neck, spine, left_arm, left_forearm, right_arm, right_forearm,
           left_leg, left_foot, right_leg, right_foot,
           W_head_neck, W_spine, W_left_arm, W_left_forearm, W_right_arm,
           W_right_forearm, W_left_leg, W_left_foot, W_right_leg, W_right_foot):
    xs = (head_neck, spine, left_arm, left_forearm, right_arm, right_forearm,
          left_leg, left_foot, right_leg, right_foot)
    ws = (W_head_neck, W_spine, W_left_arm, W_left_forearm, W_right_arm,
          W_right_forearm, W_left_leg, W_left_foot, W_right_leg, W_right_foot)

    idx_list = []
    total_loss = jnp.asarray(0.0, dtype=jnp.float32)
    for g in range(_GROUPS):
        idx, lpart = _vq_tc(xs[g], ws[g])
        idx_list.append(idx)
        total_loss = total_loss + (1.0 + _CC[g]) * lpart[0, 0] / (_B * _D)

    indices = jnp.stack(idx_list, axis=0)
    table = jnp.concatenate(ws, axis=0)
    quant_flat = _sc_gather()(indices.reshape(-1), table)
    quantized = quant_flat.reshape(_GROUPS, _B, _D)
    return quantized, total_loss, indices


# speedup vs baseline: 1.2651x; 1.2651x over previous
import functools

import jax
import jax.numpy as jnp
from jax import lax
from jax.experimental import pallas as pl
from jax.experimental.pallas import tpu as pltpu
from jax.experimental.pallas import tpu_sc as plsc

_GROUPS = 10
_B = 16384
_D = 256
_K = 1024
_CC = (0.5, 0.5, 0.4, 0.4, 0.4, 0.4, 0.8, 0.8, 0.8, 0.8)


_BB = 512
_NB = _B // _BB


def _vq_tc_body(x_ref, w_ref, idx_ref, loss_ref, w2_ref, wsq_ref):
    i = pl.program_id(0)

    @pl.when(i == 0)
    def _():
        w = w_ref[...]
        w2_ref[...] = -2.0 * w
        wsq_ref[...] = jnp.sum(w * w, axis=1)[None, :]
        loss_ref[0, 0] = 0.0

    x = x_ref[...]
    m2xw = lax.dot_general(x, w2_ref[...], (((1,), (1,)), ((), ())),
                           preferred_element_type=jnp.float32)
    xsq = jnp.sum(x * x, axis=1, keepdims=True)
    dist = (xsq + wsq_ref[...]) + m2xw
    dmin = jnp.min(dist, axis=1, keepdims=True)
    iota = lax.broadcasted_iota(jnp.int32, dist.shape, 1).astype(jnp.float32)
    idxf = jnp.min(jnp.where(dist == dmin, iota, float(_K)), axis=1)
    idx_ref[...] = idxf.astype(jnp.int32)
    loss_ref[0, 0] += jnp.sum(dmin)


def _vq_tc(x, w):
    return pl.pallas_call(
        _vq_tc_body,
        grid=(_NB,),
        in_specs=[
            pl.BlockSpec((_BB, _D), lambda i: (i, 0)),
            pl.BlockSpec((_K, _D), lambda i: (0, 0)),
        ],
        out_specs=[
            pl.BlockSpec((_BB,), lambda i: (i,)),
            pl.BlockSpec((1, 1), lambda i: (0, 0), memory_space=pltpu.SMEM),
        ],
        out_shape=[
            jax.ShapeDtypeStruct((_B,), jnp.int32),
            jax.ShapeDtypeStruct((1, 1), jnp.float32),
        ],
        scratch_shapes=[
            pltpu.VMEM((_K, _D), jnp.float32),
            pltpu.VMEM((1, _K), jnp.float32),
        ],
    )(x, w)


_NC = 2
_NS = 16
_NW = _NC * _NS
_ROWS = _GROUPS * _B
_RPW = _ROWS // _NW
_CH = 128
_NCH = _RPW // _CH


def _sc_gather_body(idx_hbm, table_hbm, out_hbm,
                    idx0, idx1, rows0, rows1, gsem, wsem0, wsem1):
    wid = lax.axis_index("s") * _NC + lax.axis_index("c")
    base = wid * _RPW
    idx_bufs = (idx0, idx1)
    row_bufs = (rows0, rows1)
    wsems = (wsem0, wsem1)

    def outer(o, _):
        for b in range(2):
            j = o * 2 + b
            start = base + j * _CH
            pltpu.sync_copy(idx_hbm.at[pl.ds(start, _CH)], idx_bufs[b])
            off = (start // _B) * _K
            for s in range(_CH // 16):
                sl = pl.ds(s * 16, 16)
                idx_bufs[b][sl] = idx_bufs[b][sl] + off

            @pl.when(j >= 2)
            def _():
                pltpu.make_async_copy(
                    row_bufs[b], out_hbm.at[pl.ds(start - 2 * _CH, _CH)],
                    wsems[b]).wait()

            pltpu.async_copy(table_hbm.at[idx_bufs[b]], row_bufs[b],
                             gsem).wait()
            pltpu.async_copy(row_bufs[b], out_hbm.at[pl.ds(start, _CH)],
                             wsems[b])
        return _

    lax.fori_loop(0, _NCH // 2, outer, None)
    for b in range(2):
        j = _NCH - 2 + b
        pltpu.make_async_copy(
            row_bufs[b], out_hbm.at[pl.ds(base + j * _CH, _CH)],
            wsems[b]).wait()


def _sc_gather():
    return pl.kernel(
        _sc_gather_body,
        mesh=plsc.VectorSubcoreMesh(core_axis_name="c", subcore_axis_name="s"),
        out_type=jax.ShapeDtypeStruct((_ROWS, _D), jnp.float32),
        scratch_types=[
            pltpu.VMEM((_CH,), jnp.int32),
            pltpu.VMEM((_CH,), jnp.int32),
            pltpu.VMEM((_CH, _D), jnp.float32),
            pltpu.VMEM((_CH, _D), jnp.float32),
            pltpu.SemaphoreType.DMA,
            pltpu.SemaphoreType.DMA,
            pltpu.SemaphoreType.DMA,
        ],
    )


def kernel(head_neck, spine, left_arm, left_forearm, right_arm, right_forearm,
           left_leg, left_foot, right_leg, right_foot,
           W_head_neck, W_spine, W_left_arm, W_left_forearm, W_right_arm,
           W_right_forearm, W_left_leg, W_left_foot, W_right_leg, W_right_foot):
    xs = (head_neck, spine, left_arm, left_forearm, right_arm, right_forearm,
          left_leg, left_foot, right_leg, right_foot)
    ws = (W_head_neck, W_spine, W_left_arm, W_left_forearm, W_right_arm,
          W_right_forearm, W_left_leg, W_left_foot, W_right_leg, W_right_foot)

    idx_list = []
    total_loss = jnp.asarray(0.0, dtype=jnp.float32)
    for g in range(_GROUPS):
        idx, lpart = _vq_tc(xs[g], ws[g])
        idx_list.append(idx)
        total_loss = total_loss + (1.0 + _CC[g]) * lpart[0, 0] / (_B * _D)

    indices = jnp.stack(idx_list, axis=0)
    table = jnp.concatenate(ws, axis=0)
    quant_flat = _sc_gather()(indices.reshape(-1), table)
    quantized = quant_flat.reshape(_GROUPS, _B, _D)
    return quantized, total_loss, indices


# speedup vs baseline: 1.4431x; 1.1408x over previous
import functools

import jax
import jax.numpy as jnp
from jax import lax
from jax.experimental import pallas as pl
from jax.experimental.pallas import tpu as pltpu
from jax.experimental.pallas import tpu_sc as plsc

_GROUPS = 10
_B = 16384
_D = 256
_K = 1024
_CC = (0.5, 0.5, 0.4, 0.4, 0.4, 0.4, 0.8, 0.8, 0.8, 0.8)


_BB = 512
_NB = _B // _BB


def _vq_tc_body(xa_ref, wa_ref, xb_ref, wb_ref,
                idxa_ref, lossa_ref, idxb_ref, lossb_ref):
    i = pl.program_id(0)

    @pl.when(i == 0)
    def _():
        lossa_ref[0, 0] = 0.0
        lossb_ref[0, 0] = 0.0

    for x_ref, w_ref, idx_ref, loss_ref in (
            (xa_ref, wa_ref, idxa_ref, lossa_ref),
            (xb_ref, wb_ref, idxb_ref, lossb_ref)):
        x = x_ref[...]
        w = w_ref[...]
        xw = lax.dot_general(x, w, (((1,), (1,)), ((), ())),
                             preferred_element_type=jnp.float32)
        xsq = jnp.sum(x * x, axis=1, keepdims=True)
        wsq = jnp.sum(w * w, axis=1)
        dist = xsq + wsq[None, :] - 2.0 * xw
        dmin = jnp.min(dist, axis=1, keepdims=True)
        iota = lax.broadcasted_iota(jnp.int32, dist.shape, 1).astype(jnp.float32)
        idxf = jnp.min(jnp.where(dist == dmin, iota, float(_K)), axis=1)
        idx_ref[...] = idxf.astype(jnp.int32)
        loss_ref[0, 0] += jnp.sum(dmin)


def _vq_tc2(xa, wa, xb, wb):
    xspec = pl.BlockSpec((_BB, _D), lambda i: (i, 0))
    wspec = pl.BlockSpec((_K, _D), lambda i: (0, 0))
    ispec = pl.BlockSpec((_BB,), lambda i: (i,))
    lspec = pl.BlockSpec((1, 1), lambda i: (0, 0), memory_space=pltpu.SMEM)
    return pl.pallas_call(
        _vq_tc_body,
        grid=(_NB,),
        in_specs=[xspec, wspec, xspec, wspec],
        out_specs=[ispec, lspec, ispec, lspec],
        out_shape=[
            jax.ShapeDtypeStruct((_B,), jnp.int32),
            jax.ShapeDtypeStruct((1, 1), jnp.float32),
            jax.ShapeDtypeStruct((_B,), jnp.int32),
            jax.ShapeDtypeStruct((1, 1), jnp.float32),
        ],
    )(xa, wa, xb, wb)


_NC = 2
_NS = 16
_NW = _NC * _NS
_ROWS = _GROUPS * _B
_RPW = _ROWS // _NW
_CH = 128
_NCH = _RPW // _CH


def _sc_gather_body(idx_hbm, table_hbm, out_hbm,
                    idx0, idx1, rows0, rows1, gsem, wsem0, wsem1):
    wid = lax.axis_index("s") * _NC + lax.axis_index("c")
    base = wid * _RPW
    idx_bufs = (idx0, idx1)
    row_bufs = (rows0, rows1)
    wsems = (wsem0, wsem1)

    def outer(o, _):
        for b in range(2):
            j = o * 2 + b
            start = base + j * _CH
            pltpu.sync_copy(idx_hbm.at[pl.ds(start, _CH)], idx_bufs[b])
            off = (start // _B) * _K
            for s in range(_CH // 16):
                sl = pl.ds(s * 16, 16)
                idx_bufs[b][sl] = idx_bufs[b][sl] + off

            @pl.when(j >= 2)
            def _():
                pltpu.make_async_copy(
                    row_bufs[b], out_hbm.at[pl.ds(start - 2 * _CH, _CH)],
                    wsems[b]).wait()

            pltpu.async_copy(table_hbm.at[idx_bufs[b]], row_bufs[b],
                             gsem).wait()
            pltpu.async_copy(row_bufs[b], out_hbm.at[pl.ds(start, _CH)],
                             wsems[b])
        return _

    lax.fori_loop(0, _NCH // 2, outer, None)
    for b in range(2):
        j = _NCH - 2 + b
        pltpu.make_async_copy(
            row_bufs[b], out_hbm.at[pl.ds(base + j * _CH, _CH)],
            wsems[b]).wait()


def _sc_gather():
    return pl.kernel(
        _sc_gather_body,
        mesh=plsc.VectorSubcoreMesh(core_axis_name="c", subcore_axis_name="s"),
        out_type=jax.ShapeDtypeStruct((_ROWS, _D), jnp.float32),
        scratch_types=[
            pltpu.VMEM((_CH,), jnp.int32),
            pltpu.VMEM((_CH,), jnp.int32),
            pltpu.VMEM((_CH, _D), jnp.float32),
            pltpu.VMEM((_CH, _D), jnp.float32),
            pltpu.SemaphoreType.DMA,
            pltpu.SemaphoreType.DMA,
            pltpu.SemaphoreType.DMA,
        ],
    )


def kernel(head_neck, spine, left_arm, left_forearm, right_arm, right_forearm,
           left_leg, left_foot, right_leg, right_foot,
           W_head_neck, W_spine, W_left_arm, W_left_forearm, W_right_arm,
           W_right_forearm, W_left_leg, W_left_foot, W_right_leg, W_right_foot):
    xs = (head_neck, spine, left_arm, left_forearm, right_arm, right_forearm,
          left_leg, left_foot, right_leg, right_foot)
    ws = (W_head_neck, W_spine, W_left_arm, W_left_forearm, W_right_arm,
          W_right_forearm, W_left_leg, W_left_foot, W_right_leg, W_right_foot)

    idx_list = []
    total_loss = jnp.asarray(0.0, dtype=jnp.float32)
    for g in range(0, _GROUPS, 2):
        idxa, la, idxb, lb = _vq_tc2(xs[g], ws[g], xs[g + 1], ws[g + 1])
        idx_list.extend((idxa, idxb))
        total_loss = (total_loss
                      + (1.0 + _CC[g]) * la[0, 0] / (_B * _D)
                      + (1.0 + _CC[g + 1]) * lb[0, 0] / (_B * _D))

    indices = jnp.stack(idx_list, axis=0)
    table = jnp.concatenate(ws, axis=0)
    quant_flat = _sc_gather()(indices.reshape(-1), table)
    quantized = quant_flat.reshape(_GROUPS, _B, _D)
    return quantized, total_loss, indices


# speedup vs baseline: 1.4719x; 1.0200x over previous
import functools

import jax
import jax.numpy as jnp
from jax import lax
from jax.experimental import pallas as pl
from jax.experimental.pallas import tpu as pltpu
from jax.experimental.pallas import tpu_sc as plsc

_GROUPS = 10
_B = 16384
_D = 256
_K = 1024
_CC = (0.5, 0.5, 0.4, 0.4, 0.4, 0.4, 0.8, 0.8, 0.8, 0.8)


_BB = 512
_NB = _B // _BB


def _vq_tc_body(n, *refs):
    i = pl.program_id(0)
    x_refs, w_refs = refs[:n], refs[n:2 * n]
    idx_refs, loss_refs = refs[2 * n:3 * n], refs[3 * n:4 * n]

    @pl.when(i == 0)
    def _():
        for loss_ref in loss_refs:
            loss_ref[0, 0] = 0.0

    for x_ref, w_ref, idx_ref, loss_ref in zip(
            x_refs, w_refs, idx_refs, loss_refs):
        x = x_ref[...]
        w = w_ref[...]
        xw = lax.dot_general(x, w, (((1,), (1,)), ((), ())),
                             preferred_element_type=jnp.float32)
        xsq = jnp.sum(x * x, axis=1, keepdims=True)
        wsq = jnp.sum(w * w, axis=1)
        dist = xsq + wsq[None, :] - 2.0 * xw
        dmin = jnp.min(dist, axis=1, keepdims=True)
        iota = lax.broadcasted_iota(jnp.int32, dist.shape, 1).astype(jnp.float32)
        idxf = jnp.min(jnp.where(dist == dmin, iota, float(_K)), axis=1)
        idx_ref[...] = idxf.astype(jnp.int32)
        loss_ref[0, 0] += jnp.sum(dmin)


def _vq_tcn(xs, ws):
    n = len(xs)
    xspec = pl.BlockSpec((_BB, _D), lambda i: (i, 0))
    wspec = pl.BlockSpec((_K, _D), lambda i: (0, 0))
    ispec = pl.BlockSpec((_BB,), lambda i: (i,))
    lspec = pl.BlockSpec((1, 1), lambda i: (0, 0), memory_space=pltpu.SMEM)
    return pl.pallas_call(
        functools.partial(_vq_tc_body, n),
        grid=(_NB,),
        in_specs=[xspec] * n + [wspec] * n,
        out_specs=[ispec] * n + [lspec] * n,
        out_shape=[jax.ShapeDtypeStruct((_B,), jnp.int32)] * n
                  + [jax.ShapeDtypeStruct((1, 1), jnp.float32)] * n,
    )(*xs, *ws)


_NC = 2
_NS = 16
_NW = _NC * _NS
_ROWS = _GROUPS * _B
_RPW = _ROWS // _NW
_CH = 128
_NCH = _RPW // _CH


def _sc_gather_body(idx_hbm, table_hbm, out_hbm,
                    idx0, idx1, rows0, rows1, gsem, wsem0, wsem1):
    wid = lax.axis_index("s") * _NC + lax.axis_index("c")
    base = wid * _RPW
    idx_bufs = (idx0, idx1)
    row_bufs = (rows0, rows1)
    wsems = (wsem0, wsem1)

    def outer(o, _):
        for b in range(2):
            j = o * 2 + b
            start = base + j * _CH
            pltpu.sync_copy(idx_hbm.at[pl.ds(start, _CH)], idx_bufs[b])
            off = (start // _B) * _K
            for s in range(_CH // 16):
                sl = pl.ds(s * 16, 16)
                idx_bufs[b][sl] = idx_bufs[b][sl] + off

            @pl.when(j >= 2)
            def _():
                pltpu.make_async_copy(
                    row_bufs[b], out_hbm.at[pl.ds(start - 2 * _CH, _CH)],
                    wsems[b]).wait()

            pltpu.async_copy(table_hbm.at[idx_bufs[b]], row_bufs[b],
                             gsem).wait()
            pltpu.async_copy(row_bufs[b], out_hbm.at[pl.ds(start, _CH)],
                             wsems[b])
        return _

    lax.fori_loop(0, _NCH // 2, outer, None)
    for b in range(2):
        j = _NCH - 2 + b
        pltpu.make_async_copy(
            row_bufs[b], out_hbm.at[pl.ds(base + j * _CH, _CH)],
            wsems[b]).wait()


def _sc_gather():
    return pl.kernel(
        _sc_gather_body,
        mesh=plsc.VectorSubcoreMesh(core_axis_name="c", subcore_axis_name="s"),
        out_type=jax.ShapeDtypeStruct((_ROWS, _D), jnp.float32),
        scratch_types=[
            pltpu.VMEM((_CH,), jnp.int32),
            pltpu.VMEM((_CH,), jnp.int32),
            pltpu.VMEM((_CH, _D), jnp.float32),
            pltpu.VMEM((_CH, _D), jnp.float32),
            pltpu.SemaphoreType.DMA,
            pltpu.SemaphoreType.DMA,
            pltpu.SemaphoreType.DMA,
        ],
    )


def kernel(head_neck, spine, left_arm, left_forearm, right_arm, right_forearm,
           left_leg, left_foot, right_leg, right_foot,
           W_head_neck, W_spine, W_left_arm, W_left_forearm, W_right_arm,
           W_right_forearm, W_left_leg, W_left_foot, W_right_leg, W_right_foot):
    xs = (head_neck, spine, left_arm, left_forearm, right_arm, right_forearm,
          left_leg, left_foot, right_leg, right_foot)
    ws = (W_head_neck, W_spine, W_left_arm, W_left_forearm, W_right_arm,
          W_right_forearm, W_left_leg, W_left_foot, W_right_leg, W_right_foot)

    idx_list = []
    total_loss = jnp.asarray(0.0, dtype=jnp.float32)
    _NG = 5
    for g0 in range(0, _GROUPS, _NG):
        outs = _vq_tcn(xs[g0:g0 + _NG], ws[g0:g0 + _NG])
        idx_list.extend(outs[:_NG])
        for k, lpart in enumerate(outs[_NG:]):
            total_loss = total_loss + (
                (1.0 + _CC[g0 + k]) * lpart[0, 0] / (_B * _D))

    indices = jnp.stack(idx_list, axis=0)
    table = jnp.concatenate(ws, axis=0)
    quant_flat = _sc_gather()(indices.reshape(-1), table)
    quantized = quant_flat.reshape(_GROUPS, _B, _D)
    return quantized, total_loss, indices
